# Initial kernel scaffold; baseline (speedup 1.0000x reference)
import jax
import jax.numpy as jnp
from jax.experimental import pallas as pl


def kernel(positions, species, edge_index, embeddings, radial_weights, W_out, b_out):
    raise NotImplementedError("write your pallas kernel here")


# speedup vs baseline: 31.5192x; 31.5192x over previous
import functools

import jax
import jax.numpy as jnp
from jax import lax
from jax.experimental import pallas as pl
from jax.experimental.pallas import tpu as pltpu
from jax.experimental.pallas import tpu_sc as plsc

N_ATOMS = 10000
N_EDGES = 320000
N_SPECIES = 4
N_CHANNELS = 16
N_RADIAL = 8
CUTOFF = 5.0

NC = 2
NS = 16
NW = NC * NS
EDGES_PAD = 327680
EDGES_PER_W = EDGES_PAD // NW
CHUNK = 2048
NCHUNK = EDGES_PER_W // CHUNK
GROUPS = CHUNK // 16
ACC_ROWS = 40064
ROWS_PER_TILE = ACC_ROWS // NS

_P_COEF = (1.0, -1.0 / 6, 1.0 / 120, -1.0 / 5040, 1.0 / 362880,
           -1.0 / 39916800, 1.0 / 6227020800, -1.0 / 1307674368000,
           1.0 / 355687428096000)
_C_COEF = (1.0, -1.0 / 2, 1.0 / 24, -1.0 / 720, 1.0 / 40320,
           -1.0 / 3628800, 1.0 / 479001600, -1.0 / 87178291200,
           1.0 / 20922789888000, -1.0 / 6402373705728000)


def _horner(coefs, u):
    acc = jnp.float32(coefs[-1])
    for c in reversed(coefs[:-1]):
        acc = acc * u + jnp.float32(c)
    return acc


N_PAD = 10048


def _sc_body(src_h, dst_h, pos_h, spec_h, zeros_h, out_h,
             pos_v, spec_v, src_v, dst_v, rows_v, vals_v, acc_sh):
    cid = lax.axis_index("c")
    sid = lax.axis_index("s")
    wid = sid * NC + cid

    pltpu.sync_copy(zeros_h.at[pl.ds(sid * ROWS_PER_TILE, ROWS_PER_TILE)],
                    acc_sh.at[pl.ds(sid * ROWS_PER_TILE, ROWS_PER_TILE)])
    pltpu.sync_copy(pos_h, pos_v)
    pltpu.sync_copy(spec_h, spec_v)
    plsc.subcore_barrier()

    iota16 = lax.iota(jnp.int32, 16)

    def group_body(g, carry):
        srcv = src_v[pl.ds(g * 16, 16)]
        dstv = dst_v[pl.ds(g * 16, 16)]
        spv = plsc.load_gather(spec_v, [srcv])
        dcl = jnp.minimum(dstv, N_ATOMS - 1)
        rx = (plsc.load_gather(pos_v, [dcl])
              - plsc.load_gather(pos_v, [srcv]))
        ry = (plsc.load_gather(pos_v, [dcl + N_PAD])
              - plsc.load_gather(pos_v, [srcv + N_PAD]))
        rz = (plsc.load_gather(pos_v, [dcl + 2 * N_PAD])
              - plsc.load_gather(pos_v, [srcv + 2 * N_PAD]))
        v = rx * rx + ry * ry + rz * rz + jnp.float32(1e-12)
        bits = plsc.bitcast(v, jnp.int32)
        y = plsc.bitcast(jnp.int32(0x5F3759DF) - (bits >> 1), jnp.float32)
        half_v = jnp.float32(0.5) * v
        for _ in range(3):
            y = y * (jnp.float32(1.5) - half_v * y * y)
        d = v * y
        x = jnp.clip(d * jnp.float32(1.0 / CUTOFF), jnp.float32(0.0),
                     jnp.float32(1.0))
        w = jnp.float32(3.14159265358979) * x
        u = w * w
        sinc = _horner(_P_COEF, u)
        cosw = _horner(_C_COEF, u)
        fcut = jnp.float32(0.5) * (cosw + jnp.float32(1.0))
        gsc = fcut / (d + jnp.float32(1e-9))
        b1 = w * sinc * gsc
        t2 = jnp.float32(2.0) * cosw
        fidx = g * 16 + iota16
        bm2 = jnp.zeros((16,), jnp.float32)
        bm1 = b1
        plsc.store_scatter(vals_v, [fidx, jnp.full((16,), 0, jnp.int32)], bm1)
        for p in range(1, N_RADIAL):
            bn = t2 * bm1 - bm2
            plsc.store_scatter(vals_v, [fidx, jnp.full((16,), p, jnp.int32)],
                               bn)
            bm2, bm1 = bm1, bn
        rows16 = dstv * N_SPECIES + spv
        j = g >> 3
        cb = (g & 7) * 16
        rows_v[j, pl.ds(cb, 16)] = rows16
        return carry

    for k in range(NCHUNK):
        base = wid * EDGES_PER_W + k * CHUNK
        pltpu.sync_copy(src_h.at[pl.ds(base, CHUNK)], src_v)
        pltpu.sync_copy(dst_h.at[pl.ds(base, CHUNK)], dst_v)
        lax.fori_loop(0, GROUPS, group_body, 0)
        for j in range(16):
            pltpu.sync_copy(vals_v.at[pl.ds(j * 128, 128)],
                            acc_sh.at[rows_v.at[j]], add=True)

    plsc.subcore_barrier()
    pltpu.sync_copy(acc_sh.at[pl.ds(sid * ROWS_PER_TILE, ROWS_PER_TILE)],
                    out_h.at[cid, pl.ds(sid * ROWS_PER_TILE, ROWS_PER_TILE)])


_sc_call = pl.kernel(
    _sc_body,
    out_type=jax.ShapeDtypeStruct((NC, ACC_ROWS, N_RADIAL), jnp.float32),
    mesh=plsc.VectorSubcoreMesh(core_axis_name="c", subcore_axis_name="s",
                                num_cores=NC, num_subcores=NS),
    compiler_params=pltpu.CompilerParams(needs_layout_passes=False,
                                         use_tc_tiling_on_sc=False),
    scratch_types=[
        pltpu.VMEM((3 * N_PAD,), jnp.float32),
        pltpu.VMEM((N_PAD,), jnp.int32),
        pltpu.VMEM((CHUNK,), jnp.int32),
        pltpu.VMEM((CHUNK,), jnp.int32),
        pltpu.VMEM((16, 128), jnp.int32),
        pltpu.VMEM((CHUNK, N_RADIAL), jnp.float32),
        pltpu.VMEM_SHARED((ACC_ROWS, N_RADIAL), jnp.float32),
    ],
)


def _tc_body(acc_ref, spec_ref, m_ref, wt_ref, emb_ref, b_ref, out_ref):
    hi = lax.Precision.HIGHEST
    a = acc_ref[0] + acc_ref[1]
    node = jnp.dot(a, m_ref[...], preferred_element_type=jnp.float32,
                   precision=hi)
    n2 = node * node
    av = jnp.dot(node, wt_ref[...], preferred_element_type=jnp.float32,
                 precision=hi)
    bv = jnp.dot(n2, wt_ref[...], preferred_element_type=jnp.float32,
                 precision=hi)
    oh = (spec_ref[...] == lax.broadcasted_iota(jnp.int32, (1, N_SPECIES), 1)
          ).astype(jnp.float32)
    center = jnp.dot(oh, emb_ref[...], preferred_element_type=jnp.float32,
                     precision=hi)
    res = jnp.sum(center * (av + center * bv), axis=1, keepdims=True)
    out_ref[...] = res + b_ref[...]


def kernel(positions, species, edge_index, embeddings, radial_weights,
           W_out, b_out):
    f32 = jnp.float32
    src = edge_index[0].astype(jnp.int32)
    dst = edge_index[1].astype(jnp.int32)
    npad = EDGES_PAD - N_EDGES
    src_p = jnp.concatenate([src, jnp.zeros((npad,), jnp.int32)])
    dst_p = jnp.concatenate([dst, jnp.full((npad,), N_ATOMS, jnp.int32)])
    zeros_acc = jnp.zeros((ACC_ROWS, N_RADIAL), f32)
    pos_flat = jnp.pad(positions.astype(f32).T,
                       ((0, 0), (0, N_PAD - N_ATOMS))).reshape(-1)
    spec_pad = jnp.pad(species.astype(jnp.int32), (0, N_PAD - N_ATOMS))

    acc = _sc_call(src_p, dst_p, pos_flat, spec_pad, zeros_acc)

    n_pad_atoms = ACC_ROWS // N_SPECIES
    accv = acc.reshape(NC, n_pad_atoms, N_SPECIES * N_RADIAL)
    spec2d = jnp.concatenate(
        [species.astype(jnp.int32),
         jnp.zeros((n_pad_atoms - N_ATOMS,), jnp.int32)]).reshape(-1, 1)
    m_mat = jnp.einsum("pr,sc->sprc", radial_weights.astype(f32),
                       embeddings.astype(f32)).reshape(
                           N_SPECIES * N_RADIAL, N_RADIAL * N_CHANNELS)
    eye_c = jnp.eye(N_CHANNELS, dtype=f32)
    wt_mat = (W_out.reshape(N_RADIAL, N_CHANNELS)[:, :, None]
              * eye_c[None, :, :]).reshape(N_RADIAL * N_CHANNELS, N_CHANNELS)
    b2d = b_out.reshape(1, 1).astype(f32)

    out = pl.pallas_call(
        _tc_body,
        out_shape=jax.ShapeDtypeStruct((n_pad_atoms, 1), f32),
    )(accv, spec2d, m_mat, wt_mat, embeddings.astype(f32), b2d)
    return out[:N_ATOMS]


# speedup vs baseline: 43.9698x; 1.3950x over previous
import jax
import jax.numpy as jnp
from jax import lax
from jax.experimental import pallas as pl
from jax.experimental.pallas import tpu as pltpu
from jax.experimental.pallas import tpu_sc as plsc

N_ATOMS = 10000
N_EDGES = 320000
N_SPECIES = 4
N_CHANNELS = 16
N_RADIAL = 8
CUTOFF = 5.0

NC = 2
NS = 16
NW = NC * NS
EDGES_PER_W = N_EDGES // NW
CHUNK = 2000
NCHUNK = EDGES_PER_W // CHUNK
GROUPS = CHUNK // 16
NSUB = 25
SUBROWS = CHUNK // NSUB
ACC_ROWS = 40064
ROWS_PER_TILE = ACC_ROWS // NS
N_PAD = 10048

_P_COEF = (1.0, -1.0 / 6, 1.0 / 120, -1.0 / 5040, 1.0 / 362880,
           -1.0 / 39916800, 1.0 / 6227020800, -1.0 / 1307674368000,
           1.0 / 355687428096000)
_C_COEF = (1.0, -1.0 / 2, 1.0 / 24, -1.0 / 720, 1.0 / 40320,
           -1.0 / 3628800, 1.0 / 479001600, -1.0 / 87178291200,
           1.0 / 20922789888000, -1.0 / 6402373705728000)


def _horner(coefs, u):
    acc = jnp.float32(coefs[-1])
    for c in reversed(coefs[:-1]):
        acc = acc * u + jnp.float32(c)
    return acc


def _sc_body(src_h, dst_h, pos_h, spec_h, zeros_h, out_h,
             pos_v, spec_v, src_v, dst_v, rows_v, vals_v, acc_sh,
             sem_in0, sem_in1, sem_sc0, sem_sc1):
    cid = lax.axis_index("c")
    sid = lax.axis_index("s")
    wid = sid * NC + cid
    sem_in = (sem_in0, sem_in1)
    sem_sc = (sem_sc0, sem_sc1)

    pltpu.sync_copy(zeros_h.at[pl.ds(sid * ROWS_PER_TILE, ROWS_PER_TILE)],
                    acc_sh.at[pl.ds(sid * ROWS_PER_TILE, ROWS_PER_TILE)])
    pltpu.sync_copy(pos_h, pos_v)
    pltpu.sync_copy(spec_h, spec_v)
    plsc.subcore_barrier()

    iota16 = lax.iota(jnp.int32, 16)

    def start_in(k):
        b = k & 1
        base = wid * EDGES_PER_W + k * CHUNK
        return (pltpu.async_copy(src_h.at[pl.ds(base, CHUNK)], src_v.at[b],
                                 sem_in[b]),
                pltpu.async_copy(dst_h.at[pl.ds(base, CHUNK)], dst_v.at[b],
                                 sem_in[b]))

    ind = [None] * NCHUNK
    scat = [None] * NCHUNK
    ind[0] = start_in(0)
    for k in range(NCHUNK):
        b = k & 1
        if k + 1 < NCHUNK:
            ind[k + 1] = start_in(k + 1)
        for dsc in ind[k]:
            dsc.wait()
        if k >= 2:
            for dsc in scat[k - 2]:
                dsc.wait()

        @plsc.parallel_loop(0, GROUPS, unroll=4)
        def _group(g):
            srcv = src_v[b, pl.ds(g * 16, 16)]
            dstv = dst_v[b, pl.ds(g * 16, 16)]
            spv = plsc.load_gather(spec_v, [srcv])
            rx = (plsc.load_gather(pos_v, [dstv])
                  - plsc.load_gather(pos_v, [srcv]))
            ry = (plsc.load_gather(pos_v, [dstv + N_PAD])
                  - plsc.load_gather(pos_v, [srcv + N_PAD]))
            rz = (plsc.load_gather(pos_v, [dstv + 2 * N_PAD])
                  - plsc.load_gather(pos_v, [srcv + 2 * N_PAD]))
            v = rx * rx + ry * ry + rz * rz + jnp.float32(1e-12)
            bits = plsc.bitcast(v, jnp.int32)
            y = plsc.bitcast(jnp.int32(0x5F3759DF) - (bits >> 1), jnp.float32)
            half_v = jnp.float32(0.5) * v
            for _ in range(3):
                y = y * (jnp.float32(1.5) - half_v * y * y)
            d = v * y
            x = jnp.clip(d * jnp.float32(1.0 / CUTOFF), jnp.float32(0.0),
                         jnp.float32(1.0))
            w = jnp.float32(3.14159265358979) * x
            u = w * w
            sinc = _horner(_P_COEF, u)
            cosw = _horner(_C_COEF, u)
            fcut = jnp.float32(0.5) * (cosw + jnp.float32(1.0))
            gsc = fcut / (d + jnp.float32(1e-9))
            b1 = w * sinc * gsc
            t2 = jnp.float32(2.0) * cosw
            fidx = g * 16 + iota16
            bm2 = jnp.zeros((16,), jnp.float32)
            bm1 = b1
            plsc.store_scatter(vals_v.at[b],
                               [fidx, jnp.full((16,), 0, jnp.int32)], bm1)
            for p in range(1, N_RADIAL):
                bn = t2 * bm1 - bm2
                plsc.store_scatter(vals_v.at[b],
                                   [fidx, jnp.full((16,), p, jnp.int32)], bn)
                bm2, bm1 = bm1, bn
            rows16 = dstv * N_SPECIES + spv
            j = g // 5
            cb = (g % 5) * 16
            rows_v[b, j, pl.ds(cb, 16)] = rows16

        scat[k] = [
            pltpu.async_copy(vals_v.at[b, pl.ds(j * SUBROWS, SUBROWS)],
                             acc_sh.at[rows_v.at[b, j]], sem_sc[b], add=True)
            for j in range(NSUB)
        ]

    for k in (NCHUNK - 2, NCHUNK - 1):
        for dsc in scat[k]:
            dsc.wait()

    plsc.subcore_barrier()
    pltpu.sync_copy(acc_sh.at[pl.ds(sid * ROWS_PER_TILE, ROWS_PER_TILE)],
                    out_h.at[cid, pl.ds(sid * ROWS_PER_TILE, ROWS_PER_TILE)])


_sc_call = pl.kernel(
    _sc_body,
    out_type=jax.ShapeDtypeStruct((NC, ACC_ROWS, N_RADIAL), jnp.float32),
    mesh=plsc.VectorSubcoreMesh(core_axis_name="c", subcore_axis_name="s",
                                num_cores=NC, num_subcores=NS),
    compiler_params=pltpu.CompilerParams(needs_layout_passes=False,
                                         use_tc_tiling_on_sc=False),
    scratch_types=[
        pltpu.VMEM((3 * N_PAD,), jnp.float32),
        pltpu.VMEM((N_PAD,), jnp.int32),
        pltpu.VMEM((2, CHUNK), jnp.int32),
        pltpu.VMEM((2, CHUNK), jnp.int32),
        pltpu.VMEM((2, NSUB, SUBROWS), jnp.int32),
        pltpu.VMEM((2, CHUNK, N_RADIAL), jnp.float32),
        pltpu.VMEM_SHARED((ACC_ROWS, N_RADIAL), jnp.float32),
        pltpu.SemaphoreType.DMA,
        pltpu.SemaphoreType.DMA,
        pltpu.SemaphoreType.DMA,
        pltpu.SemaphoreType.DMA,
    ],
)


def _tc_body(acc_ref, spec_ref, m_ref, wt_ref, emb_ref, b_ref, out_ref):
    hi = lax.Precision.HIGHEST
    a = acc_ref[0] + acc_ref[1]
    node = jnp.dot(a, m_ref[...], preferred_element_type=jnp.float32,
                   precision=hi)
    n2 = node * node
    av = jnp.dot(node, wt_ref[...], preferred_element_type=jnp.float32,
                 precision=hi)
    bv = jnp.dot(n2, wt_ref[...], preferred_element_type=jnp.float32,
                 precision=hi)
    oh = (spec_ref[...] == lax.broadcasted_iota(jnp.int32, (1, N_SPECIES), 1)
          ).astype(jnp.float32)
    center = jnp.dot(oh, emb_ref[...], preferred_element_type=jnp.float32,
                     precision=hi)
    res = jnp.sum(center * (av + center * bv), axis=1, keepdims=True)
    out_ref[...] = res + b_ref[...]


def kernel(positions, species, edge_index, embeddings, radial_weights,
           W_out, b_out):
    f32 = jnp.float32
    src = edge_index[0].astype(jnp.int32)
    dst = edge_index[1].astype(jnp.int32)
    zeros_acc = jnp.zeros((ACC_ROWS, N_RADIAL), f32)
    pos_flat = jnp.pad(positions.astype(f32).T,
                       ((0, 0), (0, N_PAD - N_ATOMS))).reshape(-1)
    spec_pad = jnp.pad(species.astype(jnp.int32), (0, N_PAD - N_ATOMS))

    acc = _sc_call(src, dst, pos_flat, spec_pad, zeros_acc)

    n_pad_atoms = ACC_ROWS // N_SPECIES
    accv = acc.reshape(NC, n_pad_atoms, N_SPECIES * N_RADIAL)
    spec2d = jnp.concatenate(
        [species.astype(jnp.int32),
         jnp.zeros((n_pad_atoms - N_ATOMS,), jnp.int32)]).reshape(-1, 1)
    m_mat = jnp.einsum("pr,sc->sprc", radial_weights.astype(f32),
                       embeddings.astype(f32)).reshape(
                           N_SPECIES * N_RADIAL, N_RADIAL * N_CHANNELS)
    eye_c = jnp.eye(N_CHANNELS, dtype=f32)
    wt_mat = (W_out.reshape(N_RADIAL, N_CHANNELS)[:, :, None]
              * eye_c[None, :, :]).reshape(N_RADIAL * N_CHANNELS, N_CHANNELS)
    b2d = b_out.reshape(1, 1).astype(f32)

    out = pl.pallas_call(
        _tc_body,
        out_shape=jax.ShapeDtypeStruct((n_pad_atoms, 1), f32),
    )(accv, spec2d, m_mat, wt_mat, embeddings.astype(f32), b2d)
    return out[:N_ATOMS]
